# baseline (speedup 1.0000x reference)
import jax
import jax.numpy as jnp
import numpy as np
from jax.experimental import pallas as pl
from jax.experimental.pallas import tpu as pltpu

_STEPS = 24
_A = 8
_H = 1024
_F32 = jnp.float32
_BF16 = jnp.bfloat16
_HI = jax.lax.Precision.HIGHEST


def _gumbel_table():
    skey = jax.random.key(42)
    rows = [
        jax.random.gumbel(jax.random.fold_in(skey, s), (1, _A), _F32)
        for s in range(_STEPS)
    ]
    return jnp.concatenate(rows, axis=0)


def _hilo(w):
    hi = w.astype(_BF16)
    lo = (w - hi.astype(_F32)).astype(_BF16)
    return hi, lo


def _prep_body(w0_ref, rows_ref, e_ref):
    e_ref[...] = jax.lax.dot_general(
        rows_ref[...], w0_ref[...], (((1,), (1,)), ((), ())),
        preferred_element_type=_F32, precision=_HI)


def _main_body(e_ref, w0_ref, w1_ref, b_ref, soft_ref, gum_ref,
               stats_ref, arch_ref):
    H = _H
    iota_a = jax.lax.broadcasted_iota(jnp.int32, (1, _A), 1)
    iota_t = jax.lax.broadcasted_iota(jnp.int32, (1, _STEPS), 1)
    b0 = b_ref[0:1, :]
    b1 = b_ref[1:2, :]
    zrow = jnp.zeros((1, H), _BF16)

    def cell(gates, c):
        i_g = gates[:, 0:H]
        f_g = gates[:, H:2 * H]
        g_g = gates[:, 2 * H:3 * H]
        o_g = gates[:, 3 * H:4 * H]
        c_new = jax.nn.sigmoid(f_g) * c + jax.nn.sigmoid(i_g) * jnp.tanh(g_g)
        h_new = jax.nn.sigmoid(o_g) * jnp.tanh(c_new)
        return h_new, c_new

    def bdot(act, w_ref):
        r = jax.lax.dot_general(
            act, w_ref[...], (((1,), (1,)), ((), ())),
            preferred_element_type=_F32)
        return r[0:1, :] + r[1:2, :]

    def step_fn(t, carry):
        x0e, h0, c0, h1, c1, lp_row, ent_row, act_row = carry
        h0h, h0l = _hilo(h0)
        a0 = jnp.concatenate([
            jnp.concatenate([h0h, h0h], axis=1),
            jnp.concatenate([h0l, zrow], axis=1)], axis=0)
        g0 = x0e + bdot(a0, w0_ref) + b0
        h0n, c0n = cell(g0, c0)
        xh, xl = _hilo(h0n)
        hh, hl = _hilo(h1)
        a1 = jnp.concatenate([
            jnp.concatenate([xh, xh, hh, hh], axis=1),
            jnp.concatenate([xl, zrow, hl, zrow], axis=1)], axis=0)
        g1 = bdot(a1, w1_ref) + b1
        h1n, c1n = cell(g1, c1)
        logits = jax.lax.dot_general(
            h1n, soft_ref[...], (((1,), (0,)), ((), ())),
            preferred_element_type=_F32, precision=_HI)
        m = jnp.max(logits)
        logp = logits - (m + jnp.log(jnp.sum(jnp.exp(logits - m))))
        z = logits + gum_ref[pl.ds(t, 1), :]
        a = jnp.min(jnp.where(z >= jnp.max(z), iota_a, _A)).astype(jnp.int32)
        onehot = iota_a == a
        lp = jnp.sum(jnp.where(onehot, logp, 0.0))
        ent = -jnp.sum(jnp.exp(logp) * logp)
        oh9 = (jax.lax.broadcasted_iota(jnp.int32, (1, 9), 1) == a + 1)
        x0e_next = jax.lax.dot_general(
            oh9.astype(_F32), e_ref[...], (((1,), (0,)), ((), ())),
            preferred_element_type=_F32, precision=_HI)
        tmask = iota_t == t
        lp_row = jnp.where(tmask, lp, lp_row)
        ent_row = jnp.where(tmask, ent, ent_row)
        act_row = jnp.where(tmask, a, act_row)
        return (x0e_next, h0n, c0n, h1n, c1n, lp_row, ent_row, act_row)

    zvec = jnp.zeros((1, H), _F32)
    init = (e_ref[0:1, :], zvec, zvec, zvec, zvec,
            jnp.zeros((1, _STEPS), _F32), jnp.zeros((1, _STEPS), _F32),
            jnp.zeros((1, _STEPS), jnp.int32))
    carry = jax.lax.fori_loop(0, _STEPS, step_fn, init)
    _, _, _, _, _, lp_row, ent_row, act_row = carry
    stats_ref[0:1, :] = lp_row
    stats_ref[1:2, :] = ent_row
    arch_ref[...] = act_row


def kernel(g_emb, w_emb, soft_emb, W_ih, W_hh, b_ih, b_hh):
    rows = jnp.concatenate([g_emb, w_emb], axis=0)
    e = pl.pallas_call(
        _prep_body,
        out_shape=jax.ShapeDtypeStruct((9, 4096), _F32),
        compiler_params=pltpu.CompilerParams(
            vmem_limit_bytes=64 * 1024 * 1024),
    )(W_ih[0], rows)

    w0h, w0l = _hilo(W_hh[0])
    w0 = jnp.concatenate([w0h, w0l], axis=1)
    w1ih, w1il = _hilo(W_ih[1])
    w1hh, w1hl = _hilo(W_hh[1])
    w1 = jnp.concatenate([w1ih, w1il, w1hh, w1hl], axis=1)
    b = b_ih + b_hh
    gum = _gumbel_table()

    stats, arch_row = pl.pallas_call(
        _main_body,
        out_shape=[
            jax.ShapeDtypeStruct((2, _STEPS), _F32),
            jax.ShapeDtypeStruct((1, _STEPS), jnp.int32),
        ],
        compiler_params=pltpu.CompilerParams(
            vmem_limit_bytes=100 * 1024 * 1024,
            allow_input_fusion=[False, True, True, False, False, False]),
    )(e, w0, w1, b, soft_emb, gum)
    return stats, arch_row[0]


# speedup vs baseline: 1.2153x; 1.2153x over previous
import jax
import jax.numpy as jnp
import numpy as np
from jax.experimental import pallas as pl
from jax.experimental.pallas import tpu as pltpu

_STEPS = 24
_A = 8
_H = 1024
_F32 = jnp.float32
_BF16 = jnp.bfloat16
_HI = jax.lax.Precision.HIGHEST


def _gumbel_table():
    skey = jax.random.key(42)
    rows = [
        jax.random.gumbel(jax.random.fold_in(skey, s), (1, _A), _F32)
        for s in range(_STEPS)
    ]
    return jnp.concatenate(rows, axis=0)


def _hilo(w):
    hi = w.astype(_BF16)
    lo = (w - hi.astype(_F32)).astype(_BF16)
    return hi, lo


def _prep_body(w0_ref, rows_ref, e_ref):
    e_ref[...] = jax.lax.dot_general(
        rows_ref[...], w0_ref[...], (((1,), (1,)), ((), ())),
        preferred_element_type=_F32, precision=_HI)


def _main_body(e_ref, w0_ref, w1_ref, b_ref, soft_ref, gum_ref,
               stats_ref, arch_ref):
    H = _H
    iota_a = jax.lax.broadcasted_iota(jnp.int32, (1, _A), 1)
    iota_t = jax.lax.broadcasted_iota(jnp.int32, (1, _STEPS), 1)
    b0 = b_ref[0:1, :]
    b1 = b_ref[1:2, :]
    zrow = jnp.zeros((1, H), _BF16)

    def cell(gates, c):
        i_g = gates[:, 0:H]
        f_g = gates[:, H:2 * H]
        g_g = gates[:, 2 * H:3 * H]
        o_g = gates[:, 3 * H:4 * H]
        c_new = jax.nn.sigmoid(f_g) * c + jax.nn.sigmoid(i_g) * jnp.tanh(g_g)
        h_new = jax.nn.sigmoid(o_g) * jnp.tanh(c_new)
        return h_new, c_new

    def bdot(act, w_ref):
        r = jax.lax.dot_general(
            act, w_ref[...], (((1,), (0,)), ((), ())),
            preferred_element_type=_F32)
        return r[0:1, :] + r[1:2, :]

    def step_fn(t, carry):
        x0e, h0, c0, h1, c1, lp_row, ent_row, act_row = carry
        h0h, h0l = _hilo(h0)
        a0 = jnp.concatenate([
            jnp.concatenate([h0h, h0h], axis=1),
            jnp.concatenate([h0l, zrow], axis=1)], axis=0)
        g0 = x0e + bdot(a0, w0_ref) + b0
        h0n, c0n = cell(g0, c0)
        xh, xl = _hilo(h0n)
        hh, hl = _hilo(h1)
        a1 = jnp.concatenate([
            jnp.concatenate([xh, xh, hh, hh], axis=1),
            jnp.concatenate([xl, zrow, hl, zrow], axis=1)], axis=0)
        g1 = bdot(a1, w1_ref) + b1
        h1n, c1n = cell(g1, c1)
        logits = jax.lax.dot_general(
            h1n, soft_ref[...], (((1,), (0,)), ((), ())),
            preferred_element_type=_F32, precision=_HI)
        m = jnp.max(logits)
        logp = logits - (m + jnp.log(jnp.sum(jnp.exp(logits - m))))
        z = logits + gum_ref[pl.ds(t, 1), :]
        a = jnp.min(jnp.where(z >= jnp.max(z), iota_a, _A)).astype(jnp.int32)
        onehot = iota_a == a
        lp = jnp.sum(jnp.where(onehot, logp, 0.0))
        ent = -jnp.sum(jnp.exp(logp) * logp)
        oh9 = (jax.lax.broadcasted_iota(jnp.int32, (1, 9), 1) == a + 1)
        x0e_next = jax.lax.dot_general(
            oh9.astype(_F32), e_ref[...], (((1,), (0,)), ((), ())),
            preferred_element_type=_F32, precision=_HI)
        tmask = iota_t == t
        lp_row = jnp.where(tmask, lp, lp_row)
        ent_row = jnp.where(tmask, ent, ent_row)
        act_row = jnp.where(tmask, a, act_row)
        return (x0e_next, h0n, c0n, h1n, c1n, lp_row, ent_row, act_row)

    zvec = jnp.zeros((1, H), _F32)
    init = (e_ref[0:1, :], zvec, zvec, zvec, zvec,
            jnp.zeros((1, _STEPS), _F32), jnp.zeros((1, _STEPS), _F32),
            jnp.zeros((1, _STEPS), jnp.int32))
    carry = jax.lax.fori_loop(0, _STEPS, step_fn, init)
    _, _, _, _, _, lp_row, ent_row, act_row = carry
    stats_ref[0:1, :] = lp_row
    stats_ref[1:2, :] = ent_row
    arch_ref[...] = act_row


def kernel(g_emb, w_emb, soft_emb, W_ih, W_hh, b_ih, b_hh):
    rows = jnp.concatenate([g_emb, w_emb], axis=0)
    e = pl.pallas_call(
        _prep_body,
        out_shape=jax.ShapeDtypeStruct((9, 4096), _F32),
        compiler_params=pltpu.CompilerParams(
            vmem_limit_bytes=64 * 1024 * 1024),
    )(W_ih[0], rows)

    w0h, w0l = _hilo(W_hh[0])
    w0 = jnp.concatenate([w0h.T, w0l.T], axis=0)
    w1ih, w1il = _hilo(W_ih[1])
    w1hh, w1hl = _hilo(W_hh[1])
    w1 = jnp.concatenate(
        [w1ih.T, w1il.T, w1hh.T, w1hl.T], axis=0)
    b = b_ih + b_hh
    gum = _gumbel_table()

    stats, arch_row = pl.pallas_call(
        _main_body,
        out_shape=[
            jax.ShapeDtypeStruct((2, _STEPS), _F32),
            jax.ShapeDtypeStruct((1, _STEPS), jnp.int32),
        ],
        compiler_params=pltpu.CompilerParams(
            vmem_limit_bytes=100 * 1024 * 1024,
            allow_input_fusion=[False, True, True, False, False, False]),
    )(e, w0, w1, b, soft_emb, gum)
    return stats, arch_row[0]


# speedup vs baseline: 1.6657x; 1.3706x over previous
import jax
import jax.numpy as jnp
import numpy as np
from jax.experimental import pallas as pl
from jax.experimental.pallas import tpu as pltpu

_STEPS = 24
_A = 8
_H = 1024
_TW = 128
_TH = 2 * _H
_TPH = _H // _TW
_TPM = 2 * _TPH
_GRID = 4 * _TPM + 1
_F32 = jnp.float32
_BF16 = jnp.bfloat16
_HI = jax.lax.Precision.HIGHEST


def _gumbel_table():
    skey = jax.random.key(42)
    rows = [
        jax.random.gumbel(jax.random.fold_in(skey, s), (1, _A), _F32)
        for s in range(_STEPS)
    ]
    return jnp.concatenate(rows, axis=0)


def _body(g_ref, emb_ref, softt_ref, wih_ref, whh_ref, bih_ref, bhh_ref,
          gum_ref, stats_ref, arch_ref, w0s, w1s, es):
    H = _H
    i = pl.program_id(0)

    def prologue(start, half, work):
        @pl.when((i >= start) & (i < start + _TPH))
        def _():
            j = i - start
            work(j * _TW, j, half * _TH)

    def e_work(ref):
        def work(row_base, j, col0):
            rows_tile = jnp.concatenate(
                [g_ref[:, pl.ds(row_base, _TW)],
                 emb_ref[:, pl.ds(row_base, _TW)]], axis=0)
            part = jax.lax.dot_general(
                ref[0], rows_tile.T, (((1,), (0,)), ((), ())),
                preferred_element_type=_F32, precision=_HI)
            prev = jnp.where(j == 0, jnp.zeros((9, _TH), _F32),
                             es[:, col0:col0 + _TH])
            es[:, col0:col0 + _TH] = prev + part.T
        return work

    def split_work(ref, out_ref, base):
        def work(row_base, j, col0):
            block = ref[0]
            hi = block.astype(_BF16)
            lo = (block - hi.astype(_F32)).astype(_BF16)
            out_ref[pl.ds(base + row_base, _TW), col0:col0 + _TH] = hi.T
            out_ref[pl.ds(base + H + row_base, _TW), col0:col0 + _TH] = lo.T
        return work

    for half in (0, 1):
        prologue(0 * _TPM + half * _TPH, half, e_work(wih_ref))
        prologue(1 * _TPM + half * _TPH, half, split_work(whh_ref, w0s, 0))
        prologue(2 * _TPM + half * _TPH, half, split_work(wih_ref, w1s, 0))
        prologue(3 * _TPM + half * _TPH, half,
                 split_work(whh_ref, w1s, 2 * H))

    @pl.when(i == _GRID - 1)
    def _():
        iota_a = jax.lax.broadcasted_iota(jnp.int32, (1, _A), 1)
        iota_t = jax.lax.broadcasted_iota(jnp.int32, (1, _STEPS), 1)
        b0 = bih_ref[0:1, :] + bhh_ref[0:1, :]
        b1 = bih_ref[1:2, :] + bhh_ref[1:2, :]
        soft = softt_ref[...].T
        zrow = jnp.zeros((1, H), _BF16)

        def hilo(x):
            hi = x.astype(_BF16)
            lo = (x - hi.astype(_F32)).astype(_BF16)
            return hi, lo

        def cell(gates, c):
            i_g = gates[:, 0:H]
            f_g = gates[:, H:2 * H]
            g_g = gates[:, 2 * H:3 * H]
            o_g = gates[:, 3 * H:4 * H]
            c_new = (jax.nn.sigmoid(f_g) * c
                     + jax.nn.sigmoid(i_g) * jnp.tanh(g_g))
            h_new = jax.nn.sigmoid(o_g) * jnp.tanh(c_new)
            return h_new, c_new

        def bdot(act, w_ref):
            r = jax.lax.dot_general(
                act, w_ref[...], (((1,), (0,)), ((), ())),
                preferred_element_type=_F32)
            return r[0:1, :] + r[1:2, :]

        def step_fn(t, carry):
            x0e, h0, c0, h1, c1, lp_row, ent_row, act_row = carry
            h0h, h0l = hilo(h0)
            a0 = jnp.concatenate([
                jnp.concatenate([h0h, h0h], axis=1),
                jnp.concatenate([h0l, zrow], axis=1)], axis=0)
            g0 = x0e + bdot(a0, w0s) + b0
            h0n, c0n = cell(g0, c0)
            xh, xl = hilo(h0n)
            hh, hl = hilo(h1)
            a1 = jnp.concatenate([
                jnp.concatenate([xh, xh, hh, hh], axis=1),
                jnp.concatenate([xl, zrow, hl, zrow], axis=1)],
                axis=0)
            g1 = bdot(a1, w1s) + b1
            h1n, c1n = cell(g1, c1)
            logits = jax.lax.dot_general(
                h1n, soft, (((1,), (0,)), ((), ())),
                preferred_element_type=_F32, precision=_HI)
            m = jnp.max(logits)
            logp = logits - (m + jnp.log(jnp.sum(jnp.exp(logits - m))))
            z = logits + gum_ref[pl.ds(t, 1), :]
            a = jnp.min(
                jnp.where(z >= jnp.max(z), iota_a, _A)).astype(jnp.int32)
            onehot = iota_a == a
            lp = jnp.sum(jnp.where(onehot, logp, 0.0))
            ent = -jnp.sum(jnp.exp(logp) * logp)
            oh9 = (jax.lax.broadcasted_iota(jnp.int32, (1, 9), 1) == a + 1)
            x0e_next = jax.lax.dot_general(
                oh9.astype(_F32), es[...], (((1,), (0,)), ((), ())),
                preferred_element_type=_F32, precision=_HI)
            tmask = iota_t == t
            lp_row = jnp.where(tmask, lp, lp_row)
            ent_row = jnp.where(tmask, ent, ent_row)
            act_row = jnp.where(tmask, a, act_row)
            return (x0e_next, h0n, c0n, h1n, c1n, lp_row, ent_row, act_row)

        zvec = jnp.zeros((1, H), _F32)
        init = (es[0:1, :], zvec, zvec, zvec, zvec,
                jnp.zeros((1, _STEPS), _F32), jnp.zeros((1, _STEPS), _F32),
                jnp.zeros((1, _STEPS), jnp.int32))
        carry = jax.lax.fori_loop(0, _STEPS, step_fn, init)
        _, _, _, _, _, lp_row, ent_row, act_row = carry
        stats_ref[0:1, :] = lp_row
        stats_ref[1:2, :] = ent_row
        arch_ref[...] = act_row


def _wih_index(i):
    l = jnp.where(i < 2 * _TPM, 0, 1)
    s = jnp.clip(jnp.where(i < 2 * _TPM, i, i - 2 * _TPM), 0, _TPM - 1)
    return (l, s // _TPH, s % _TPH)


def _whh_index(i):
    l = jnp.where(i < 3 * _TPM, 0, 1)
    s = jnp.clip(jnp.where(i < 3 * _TPM, i - _TPM, i - 3 * _TPM),
                 0, _TPM - 1)
    return (l, s // _TPH, s % _TPH)


def _full(shape):
    return pl.BlockSpec(shape, lambda i: tuple(0 for _ in shape))


def kernel(g_emb, w_emb, soft_emb, W_ih, W_hh, b_ih, b_hh):
    gum = _gumbel_table()
    stats, arch_row = pl.pallas_call(
        _body,
        grid=(_GRID,),
        in_specs=[
            _full((1, _H)),
            _full((_A, _H)),
            _full((_A, _H)),
            pl.BlockSpec((1, _TH, _TW), _wih_index),
            pl.BlockSpec((1, _TH, _TW), _whh_index),
            _full((2, 4 * _H)),
            _full((2, 4 * _H)),
            _full((_STEPS, _A)),
        ],
        out_specs=[
            _full((2, _STEPS)),
            _full((1, _STEPS)),
        ],
        out_shape=[
            jax.ShapeDtypeStruct((2, _STEPS), _F32),
            jax.ShapeDtypeStruct((1, _STEPS), jnp.int32),
        ],
        scratch_shapes=[
            pltpu.VMEM((2 * _H, 4 * _H), _BF16),
            pltpu.VMEM((4 * _H, 4 * _H), _BF16),
            pltpu.VMEM((9, 4 * _H), _F32),
        ],
        compiler_params=pltpu.CompilerParams(
            dimension_semantics=("arbitrary",),
            vmem_limit_bytes=63 * 1024 * 1024),
    )(g_emb, w_emb, soft_emb.T, W_ih, W_hh, b_ih, b_hh, gum)
    return stats, arch_row[0]


# speedup vs baseline: 1.7329x; 1.0404x over previous
import jax
import jax.numpy as jnp
import numpy as np
from jax.experimental import pallas as pl
from jax.experimental.pallas import tpu as pltpu

_STEPS = 24
_A = 8
_H = 1024
_TW = 128
_TH = 2 * _H
_TPH = _H // _TW
_TPM = 2 * _TPH
_GRID = 4 * _TPM + 1
_F32 = jnp.float32
_BF16 = jnp.bfloat16
_HI = jax.lax.Precision.HIGHEST


def _gumbel_table():
    skey = jax.random.key(42)
    rows = [
        jax.random.gumbel(jax.random.fold_in(skey, s), (1, _A), _F32)
        for s in range(_STEPS)
    ]
    return jnp.concatenate(rows, axis=0)


def _body(g_ref, emb_ref, softt_ref, wih_ref, whh_ref, bih_ref, bhh_ref,
          gum_ref, stats_ref, arch_ref, w0s, w1s, es, ls):
    H = _H
    i = pl.program_id(0)

    def prologue(start, half, work):
        @pl.when((i >= start) & (i < start + _TPH))
        def _():
            j = i - start
            work(j * _TW, j, half * _TH)

    def e_work(ref):
        def work(row_base, j, col0):
            rows_tile = jnp.concatenate(
                [g_ref[:, pl.ds(row_base, _TW)],
                 emb_ref[:, pl.ds(row_base, _TW)]], axis=0)
            part = jax.lax.dot_general(
                ref[0], rows_tile.T, (((1,), (0,)), ((), ())),
                preferred_element_type=_F32, precision=_HI)
            prev = jnp.where(j == 0, jnp.zeros((9, _TH), _F32),
                             es[:, col0:col0 + _TH])
            es[:, col0:col0 + _TH] = prev + part.T
        return work

    def split_work(ref, out_ref, base):
        def work(row_base, j, col0):
            block = ref[0]
            hi = block.astype(_BF16)
            lo = (block - hi.astype(_F32)).astype(_BF16)
            out_ref[pl.ds(base + row_base, _TW), col0:col0 + _TH] = hi.T
            out_ref[pl.ds(base + H + row_base, _TW), col0:col0 + _TH] = lo.T
        return work

    for half in (0, 1):
        prologue(0 * _TPM + half * _TPH, half, e_work(wih_ref))
        prologue(1 * _TPM + half * _TPH, half, split_work(whh_ref, w0s, 0))
        prologue(2 * _TPM + half * _TPH, half, split_work(wih_ref, w1s, 0))
        prologue(3 * _TPM + half * _TPH, half,
                 split_work(whh_ref, w1s, 2 * H))

    @pl.when(i == _GRID - 1)
    def _():
        iota_a = jax.lax.broadcasted_iota(jnp.int32, (1, _A), 1)
        iota_t = jax.lax.broadcasted_iota(jnp.int32, (1, _STEPS), 1)
        b0 = bih_ref[0:1, :] + bhh_ref[0:1, :]
        b1 = bih_ref[1:2, :] + bhh_ref[1:2, :]
        soft = softt_ref[...].T

        def hilo2(x):
            hi = x.astype(_BF16)
            lo = (x - hi.astype(_F32)).astype(_BF16)
            return jnp.concatenate([
                jnp.concatenate([hi, hi], axis=1),
                jnp.concatenate([lo, jnp.zeros((1, H), _BF16)], axis=1)],
                axis=0)

        def cell(gates, c):
            i_g = gates[:, 0:H]
            f_g = gates[:, H:2 * H]
            g_g = gates[:, 2 * H:3 * H]
            o_g = gates[:, 3 * H:4 * H]
            c_new = (jax.nn.sigmoid(f_g) * c
                     + jax.nn.sigmoid(i_g) * jnp.tanh(g_g))
            h_new = jax.nn.sigmoid(o_g) * jnp.tanh(c_new)
            return h_new, c_new

        def bdot(act, wv):
            r = jax.lax.dot_general(
                act, wv, (((1,), (0,)), ((), ())),
                preferred_element_type=_F32)
            return r[0:1, :] + r[1:2, :]

        def step_fn(t, carry):
            x0e, d0, c0, d1h, c1, act_row = carry
            g0 = x0e + d0 + b0
            h0n, c0n = cell(g0, c0)
            a0n = hilo2(h0n)
            d0n = bdot(a0n, w0s[...])
            d1x = bdot(a0n, w1s[0:2 * H, :])
            g1 = d1x + d1h + b1
            h1n, c1n = cell(g1, c1)
            d1hn = bdot(hilo2(h1n), w1s[2 * H:4 * H, :])
            logits = jax.lax.dot_general(
                h1n, soft, (((1,), (0,)), ((), ())),
                preferred_element_type=_F32, precision=_HI)
            ls[pl.ds(t, 1), :] = logits
            z = logits + gum_ref[pl.ds(t, 1), :]
            a = jnp.min(
                jnp.where(z >= jnp.max(z), iota_a, _A)).astype(jnp.int32)
            x0e_next = es[pl.ds(a + 1, 1), :]
            act_row = jnp.where(iota_t == t, a, act_row)
            return (x0e_next, d0n, c0n, d1hn, c1n, act_row)

        zvec = jnp.zeros((1, 4 * H), _F32)
        init = (es[0:1, :], zvec, jnp.zeros((1, H), _F32),
                zvec, jnp.zeros((1, H), _F32),
                jnp.zeros((1, _STEPS), jnp.int32))
        carry = jax.lax.fori_loop(0, _STEPS, step_fn, init)
        act_row = carry[5]

        L = ls[...]
        m = jnp.max(L, axis=1, keepdims=True)
        logp = L - (m + jnp.log(jnp.sum(jnp.exp(L - m), axis=1,
                                        keepdims=True)))
        ent_col = -jnp.sum(jnp.exp(logp) * logp, axis=1, keepdims=True)
        oh = (jax.lax.broadcasted_iota(jnp.int32, (_STEPS, _A), 1)
              == act_row.T)
        lp_col = jnp.sum(jnp.where(oh, logp, 0.0), axis=1, keepdims=True)
        stats_ref[0:1, :] = lp_col.T
        stats_ref[1:2, :] = ent_col.T
        arch_ref[...] = act_row


def _wih_index(i):
    l = jnp.where(i < 2 * _TPM, 0, 1)
    s = jnp.clip(jnp.where(i < 2 * _TPM, i, i - 2 * _TPM), 0, _TPM - 1)
    return (l, s // _TPH, s % _TPH)


def _whh_index(i):
    l = jnp.where(i < 3 * _TPM, 0, 1)
    s = jnp.clip(jnp.where(i < 3 * _TPM, i - _TPM, i - 3 * _TPM),
                 0, _TPM - 1)
    return (l, s // _TPH, s % _TPH)


def _full(shape):
    return pl.BlockSpec(shape, lambda i: tuple(0 for _ in shape))


def kernel(g_emb, w_emb, soft_emb, W_ih, W_hh, b_ih, b_hh):
    gum = _gumbel_table()
    stats, arch_row = pl.pallas_call(
        _body,
        grid=(_GRID,),
        in_specs=[
            _full((1, _H)),
            _full((_A, _H)),
            _full((_A, _H)),
            pl.BlockSpec((1, _TH, _TW), _wih_index),
            pl.BlockSpec((1, _TH, _TW), _whh_index),
            _full((2, 4 * _H)),
            _full((2, 4 * _H)),
            _full((_STEPS, _A)),
        ],
        out_specs=[
            _full((2, _STEPS)),
            _full((1, _STEPS)),
        ],
        out_shape=[
            jax.ShapeDtypeStruct((2, _STEPS), _F32),
            jax.ShapeDtypeStruct((1, _STEPS), jnp.int32),
        ],
        scratch_shapes=[
            pltpu.VMEM((2 * _H, 4 * _H), _BF16),
            pltpu.VMEM((4 * _H, 4 * _H), _BF16),
            pltpu.VMEM((9, 4 * _H), _F32),
            pltpu.VMEM((_STEPS, _A), _F32),
        ],
        compiler_params=pltpu.CompilerParams(
            dimension_semantics=("arbitrary",),
            vmem_limit_bytes=63 * 1024 * 1024),
    )(g_emb, w_emb, soft_emb.T, W_ih, W_hh, b_ih, b_hh, gum)
    return stats, arch_row[0]
